# initial kernel scaffold (unmeasured)
import jax
import jax.numpy as jnp
from jax import lax
from jax.experimental import pallas as pl
from jax.experimental.pallas import tpu as pltpu

N_DEV = 8


def _gelu(y):
    c = 0.7978845608028654
    return 0.5 * y * (1.0 + jnp.tanh(c * (y + 0.044715 * y * y * y)))


def kernel(x, w_mat):
    m_per, k = x.shape
    _, n = w_mat.shape
    n_per = n // N_DEV

    def body(x_ref, w_ref, out_ref, send_buf, recv_buf, send_sems, recv_sems):
        me = lax.axis_index("i")

        barrier_sem = pltpu.get_barrier_semaphore()
        for d in range(1, N_DEV):
            pl.semaphore_signal(
                barrier_sem, inc=1,
                device_id=((me + d) % N_DEV,),
                device_id_type=pl.DeviceIdType.MESH,
            )
        pl.semaphore_wait(barrier_sem, N_DEV - 1)

        x_bf = x_ref[...].astype(jnp.bfloat16)

        rdmas = []
        for d in range(1, N_DEV):
            t = (me + d) % N_DEV
            w_blk = w_ref[:, pl.ds(t * n_per, n_per)].astype(jnp.bfloat16)
            send_buf[d - 1, :, :] = jnp.dot(
                x_bf, w_blk, preferred_element_type=jnp.float32
            ).astype(jnp.bfloat16)
            rdma = pltpu.make_async_remote_copy(
                src_ref=send_buf.at[d - 1],
                dst_ref=recv_buf.at[d - 1],
                send_sem=send_sems.at[d - 1],
                recv_sem=recv_sems.at[d - 1],
                device_id=(t,),
                device_id_type=pl.DeviceIdType.MESH,
            )
            rdma.start()
            rdmas.append(rdma)

        w_blk = w_ref[:, pl.ds(me * n_per, n_per)].astype(jnp.bfloat16)
        y_own = jnp.dot(x_bf, w_blk, preferred_element_type=jnp.float32)
        out_ref[pl.ds(me * m_per, m_per), :] = _gelu(y_own)

        for d in range(1, N_DEV):
            rdmas[d - 1].wait_recv()
            s = (me - d) % N_DEV
            y = recv_buf[d - 1, :, :].astype(jnp.float32)
            out_ref[pl.ds(s * m_per, m_per), :] = _gelu(y)

        for d in range(1, N_DEV):
            rdmas[d - 1].wait_send()

    return pl.pallas_call(
        body,
        out_shape=jax.ShapeDtypeStruct((N_DEV * m_per, n_per), jnp.float32),
        in_specs=[
            pl.BlockSpec(memory_space=pltpu.VMEM),
            pl.BlockSpec(memory_space=pltpu.VMEM),
        ],
        out_specs=pl.BlockSpec(memory_space=pltpu.VMEM),
        scratch_shapes=[
            pltpu.VMEM((N_DEV - 1, m_per, n_per), jnp.bfloat16),
            pltpu.VMEM((N_DEV - 1, m_per, n_per), jnp.bfloat16),
            pltpu.SemaphoreType.DMA((N_DEV - 1,)),
            pltpu.SemaphoreType.DMA((N_DEV - 1,)),
        ],
        compiler_params=pltpu.CompilerParams(collective_id=0),
    )(x, w_mat)


# baseline (device time: 45978 ns/iter reference)
import jax
import jax.numpy as jnp
from jax import lax
from jax.experimental import pallas as pl
from jax.experimental.pallas import tpu as pltpu

N_DEV = 8


def _gelu(y):
    c = 0.7978845608028654
    return 0.5 * y * (1.0 + jnp.tanh(c * (y + 0.044715 * y * y * y)))


def kernel(x, w_mat):
    m_per, k = x.shape
    _, n = w_mat.shape
    n_per = n // N_DEV

    def body(x_ref, w_ref, out_ref, send_buf, recv_buf, send_sems, recv_sems):
        me = lax.axis_index("i")

        barrier_sem = pltpu.get_barrier_semaphore()
        for d in range(1, N_DEV):
            pl.semaphore_signal(
                barrier_sem, inc=1,
                device_id=((me + d) % N_DEV,),
                device_id_type=pl.DeviceIdType.MESH,
            )
        pl.semaphore_wait(barrier_sem, N_DEV - 1)

        x_bf = x_ref[...].astype(jnp.bfloat16)

        rdmas = []
        for d in range(1, N_DEV):
            t = (me + d) % N_DEV
            w_blk = w_ref[:, pl.ds(t * n_per, n_per)].astype(jnp.bfloat16)
            send_buf[d - 1, :, :] = jnp.dot(
                x_bf, w_blk, preferred_element_type=jnp.float32
            ).astype(jnp.bfloat16)
            rdma = pltpu.make_async_remote_copy(
                src_ref=send_buf.at[d - 1],
                dst_ref=recv_buf.at[d - 1],
                send_sem=send_sems.at[d - 1],
                recv_sem=recv_sems.at[d - 1],
                device_id=(t,),
                device_id_type=pl.DeviceIdType.MESH,
            )
            rdma.start()
            rdmas.append(rdma)

        w_blk = w_ref[:, pl.ds(me * n_per, n_per)].astype(jnp.bfloat16)
        y_own = jnp.dot(x_bf, w_blk, preferred_element_type=jnp.float32)
        out_ref[pl.ds(me * m_per, m_per), :] = _gelu(y_own)

        for d in range(1, N_DEV):
            rdmas[d - 1].wait_recv()
            s = (me - d) % N_DEV
            y = recv_buf[d - 1, :, :].astype(jnp.float32)
            out_ref[pl.ds(s * m_per, m_per), :] = _gelu(y)

        for d in range(1, N_DEV):
            rdmas[d - 1].wait_send()

    return pl.pallas_call(
        body,
        out_shape=jax.ShapeDtypeStruct((N_DEV * m_per, n_per), jnp.float32),
        in_specs=[
            pl.BlockSpec(memory_space=pltpu.VMEM),
            pl.BlockSpec(memory_space=pltpu.VMEM),
        ],
        out_specs=pl.BlockSpec(memory_space=pltpu.VMEM),
        scratch_shapes=[
            pltpu.VMEM((N_DEV - 1, m_per, n_per), jnp.bfloat16),
            pltpu.VMEM((N_DEV - 1, m_per, n_per), jnp.bfloat16),
            pltpu.SemaphoreType.DMA((N_DEV - 1,)),
            pltpu.SemaphoreType.DMA((N_DEV - 1,)),
        ],
        compiler_params=pltpu.CompilerParams(
            collective_id=0, vmem_limit_bytes=100 * 1024 * 1024
        ),
    )(x, w_mat)


# device time: 40433 ns/iter; 1.1371x vs baseline; 1.1371x over previous
import os

import jax
import jax.numpy as jnp
from jax import lax
from jax.experimental import pallas as pl
from jax.experimental.pallas import tpu as pltpu

N_DEV = 8
_VARIANT = os.environ.get("KVARIANT", "full")


def _gelu(y):
    c = 0.7978845608028654
    return 0.5 * y * (1.0 + jnp.tanh(c * (y + 0.044715 * y * y * y)))


def kernel(x, w_mat):
    m_per, k = x.shape
    _, n = w_mat.shape
    n_per = n // N_DEV

    def body(x_ref, w_ref, out_ref, send_buf, recv_buf, send_sems, recv_sems):
        me = lax.axis_index("i")
        use_comm = _VARIANT != "nocomm"
        use_dot = _VARIANT != "nodot"

        if use_comm:
            barrier_sem = pltpu.get_barrier_semaphore()
            for d in range(1, N_DEV):
                pl.semaphore_signal(
                    barrier_sem, inc=1,
                    device_id=((me + d) % N_DEV,),
                    device_id_type=pl.DeviceIdType.MESH,
                )
            pl.semaphore_wait(barrier_sem, N_DEV - 1)

        x_bf = x_ref[...].astype(jnp.bfloat16)

        def blk(t):
            if not use_dot:
                return x_bf[:, : n_per].astype(jnp.float32)
            w_blk = w_ref[:, pl.ds(t * n_per, n_per)].astype(jnp.bfloat16)
            return jnp.dot(x_bf, w_blk, preferred_element_type=jnp.float32)

        rdmas = []
        for d in range(1, N_DEV):
            t = (me + d) % N_DEV
            send_buf[d - 1, :, :] = blk(t).astype(jnp.bfloat16)
            if use_comm:
                rdma = pltpu.make_async_remote_copy(
                    src_ref=send_buf.at[d - 1],
                    dst_ref=recv_buf.at[d - 1],
                    send_sem=send_sems.at[d - 1],
                    recv_sem=recv_sems.at[d - 1],
                    device_id=(t,),
                    device_id_type=pl.DeviceIdType.MESH,
                )
                rdma.start()
                rdmas.append(rdma)

        out_ref[pl.ds(me * m_per, m_per), :] = _gelu(blk(me))

        for d in range(1, N_DEV):
            if use_comm:
                rdmas[d - 1].wait_recv()
            s = (me - d) % N_DEV
            src = recv_buf if use_comm else send_buf
            y = src[d - 1, :, :].astype(jnp.float32)
            out_ref[pl.ds(s * m_per, m_per), :] = _gelu(y)

        if use_comm:
            for d in range(1, N_DEV):
                rdmas[d - 1].wait_send()

    return pl.pallas_call(
        body,
        out_shape=jax.ShapeDtypeStruct((N_DEV * m_per, n_per), jnp.float32),
        in_specs=[
            pl.BlockSpec(memory_space=pltpu.VMEM),
            pl.BlockSpec(memory_space=pltpu.VMEM),
        ],
        out_specs=pl.BlockSpec(memory_space=pltpu.VMEM),
        scratch_shapes=[
            pltpu.VMEM((N_DEV - 1, m_per, n_per), jnp.bfloat16),
            pltpu.VMEM((N_DEV - 1, m_per, n_per), jnp.bfloat16),
            pltpu.SemaphoreType.DMA((N_DEV - 1,)),
            pltpu.SemaphoreType.DMA((N_DEV - 1,)),
        ],
        compiler_params=pltpu.CompilerParams(
            collective_id=0 if _VARIANT != "nocomm" else None,
            vmem_limit_bytes=100 * 1024 * 1024,
        ),
    )(x, w_mat)


# device time: 28424 ns/iter; 1.6176x vs baseline; 1.4225x over previous
import os

import jax
import jax.numpy as jnp
from jax import lax
from jax.experimental import pallas as pl
from jax.experimental.pallas import tpu as pltpu

N_DEV = 8
_VARIANT = os.environ.get("KVARIANT", "full")


def _gelu(y):
    c = 0.7978845608028654
    return 0.5 * y * (1.0 + jnp.tanh(c * (y + 0.044715 * y * y * y)))


def kernel(x, w_mat):
    m_per, k = x.shape
    _, n = w_mat.shape
    n_per = n // N_DEV
    use_comm = _VARIANT != "nocomm"

    def body(x_hbm, w_hbm, out_ref, x_vmem, w_buf, send_buf, recv_buf,
             copy_sems, send_sems, recv_sems):
        me = lax.axis_index("i")

        x_copy = pltpu.make_async_copy(x_hbm, x_vmem, copy_sems.at[2])
        x_copy.start()

        def w_copy(d, slot):
            t = (me + d) % N_DEV
            return pltpu.make_async_copy(
                w_hbm.at[:, pl.ds(t * n_per, n_per)],
                w_buf.at[slot],
                copy_sems.at[slot],
            )

        w_copy(1, 0).start()

        if use_comm:
            barrier_sem = pltpu.get_barrier_semaphore()
            for d in range(1, N_DEV):
                pl.semaphore_signal(
                    barrier_sem, inc=1,
                    device_id=((me + d) % N_DEV,),
                    device_id_type=pl.DeviceIdType.MESH,
                )
            pl.semaphore_wait(barrier_sem, N_DEV - 1)

        x_copy.wait()
        x_bf = x_vmem[...].astype(jnp.bfloat16)

        rdmas = []
        for d in range(1, N_DEV + 1):
            slot = (d - 1) % 2
            if d < N_DEV:
                w_copy(d + 1, 1 - slot).start()
            w_copy(d, slot).wait()
            w_blk = w_buf[slot].astype(jnp.bfloat16)
            y_blk = jnp.dot(x_bf, w_blk, preferred_element_type=jnp.float32)
            if d == N_DEV:
                out_ref[pl.ds(me * m_per, m_per), :] = _gelu(y_blk)
                break
            send_buf[d - 1, :, :] = y_blk.astype(jnp.bfloat16)
            if use_comm:
                t = (me + d) % N_DEV
                rdma = pltpu.make_async_remote_copy(
                    src_ref=send_buf.at[d - 1],
                    dst_ref=recv_buf.at[d - 1],
                    send_sem=send_sems.at[d - 1],
                    recv_sem=recv_sems.at[d - 1],
                    device_id=(t,),
                    device_id_type=pl.DeviceIdType.MESH,
                )
                rdma.start()
                rdmas.append(rdma)

        for d in range(1, N_DEV):
            if use_comm:
                rdmas[d - 1].wait_recv()
            s = (me - d) % N_DEV
            src = recv_buf if use_comm else send_buf
            y = src[d - 1, :, :].astype(jnp.float32)
            out_ref[pl.ds(s * m_per, m_per), :] = _gelu(y)

        if use_comm:
            for d in range(1, N_DEV):
                rdmas[d - 1].wait_send()

    return pl.pallas_call(
        body,
        out_shape=jax.ShapeDtypeStruct((N_DEV * m_per, n_per), jnp.float32),
        in_specs=[
            pl.BlockSpec(memory_space=pl.ANY),
            pl.BlockSpec(memory_space=pl.ANY),
        ],
        out_specs=pl.BlockSpec(memory_space=pltpu.VMEM),
        scratch_shapes=[
            pltpu.VMEM((m_per, k), jnp.float32),
            pltpu.VMEM((2, k, n_per), jnp.float32),
            pltpu.VMEM((N_DEV - 1, m_per, n_per), jnp.bfloat16),
            pltpu.VMEM((N_DEV - 1, m_per, n_per), jnp.bfloat16),
            pltpu.SemaphoreType.DMA((3,)),
            pltpu.SemaphoreType.DMA((N_DEV - 1,)),
            pltpu.SemaphoreType.DMA((N_DEV - 1,)),
        ],
        compiler_params=pltpu.CompilerParams(
            collective_id=0 if use_comm else None,
            vmem_limit_bytes=100 * 1024 * 1024,
        ),
    )(x, w_mat)
